# baseline (speedup 1.0000x reference)
import jax
import jax.numpy as jnp
from jax import lax
from jax.experimental import pallas as pl
from jax.experimental.pallas import tpu as pltpu
from jax.experimental.pallas import tpu_sc as plsc

_V = 100000
_E = 64
_B = 1024
_L = 16
_NC, _NS = 2, 16
_NW = _NC * _NS
_BPW = _B // _NW
_NRING = 12


def _rsqrt(x):
    i = lax.bitcast_convert_type(x, jnp.int32)
    y = lax.bitcast_convert_type(jnp.int32(0x5F3759DF) - (i >> 1), jnp.float32)
    for _ in range(3):
        y = y * (1.5 - 0.5 * x * y * y)
    return y


def _body(x_hbm, w_hbm, b_hbm, out_hbm, x_v, blk_v, b_v, out_v, *sems):
    wid = lax.axis_index("s") * _NC + lax.axis_index("c")
    base = wid * _BPW

    pltpu.sync_copy(x_hbm.at[pl.ds(base, _BPW)], x_v)
    pltpu.sync_copy(b_hbm, b_v)

    xv0 = x_v[pl.ds(0, _L)]
    xv1 = x_v[pl.ds(_L, _L)]
    xs = [xv0[i] for i in range(_L)] + [xv1[i] for i in range(_L)]
    bv = [b_v[pl.ds(k * _L, _L)] for k in range(_E // _L)]
    iota = lax.iota(jnp.int32, _L)

    def fire(i):
        ci = pl.multiple_of((xs[i] >> 7) << 7, 128)
        return pltpu.async_copy(
            w_hbm.at[:, pl.ds(ci, 128)], blk_v.at[i % _NRING], sems[i % _NRING]
        )

    copies = {}
    for i in range(_NRING):
        copies[i] = fire(i)

    for i in range(_BPW):
        copies[i].wait()
        s = i % _NRING
        li = jnp.full((_L,), xs[i] & 127, jnp.int32)
        si = jnp.full((_L,), s, jnp.int32)
        ck = [
            plsc.load_gather(blk_v, [si, k * _L + iota, li]) + bv[k]
            for k in range(_E // _L)
        ]
        if i + _NRING < _BPW:
            copies[i + _NRING] = fire(i + _NRING)
        ss = jnp.zeros((), jnp.float32)
        for c in ck:
            ss = ss + lax.reduce_sum_p.bind(c * c, axes=(0,))
        r = _rsqrt(jnp.maximum(ss, 1e-24))
        ii = jnp.full((_L,), i, jnp.int32)
        for k in range(_E // _L):
            plsc.store_scatter(out_v, [ii, k * _L + iota], ck[k] * r)

    pltpu.sync_copy(out_v, out_hbm.at[pl.ds(base, _BPW)])


@jax.jit
def _road2vec_sc(x, w, b):
    mesh = plsc.VectorSubcoreMesh(core_axis_name="c", subcore_axis_name="s")
    return pl.kernel(
        _body,
        mesh=mesh,
        compiler_params=pltpu.CompilerParams(needs_layout_passes=False),
        out_type=jax.ShapeDtypeStruct((_B, _E), jnp.float32),
        scratch_types=[
            pltpu.VMEM((_BPW,), jnp.int32),
            pltpu.VMEM((_NRING, _E, 128), jnp.float32),
            pltpu.VMEM((_E,), jnp.float32),
            pltpu.VMEM((_BPW, _E), jnp.float32),
        ]
        + [pltpu.SemaphoreType.DMA] * _NRING,
    )(x, w, b)


def kernel(x, W, b):
    return _road2vec_sc(x.astype(jnp.int32), W, b)


# speedup vs baseline: 1.0305x; 1.0305x over previous
import jax
import jax.numpy as jnp
from jax import lax
from jax.experimental import pallas as pl
from jax.experimental.pallas import tpu as pltpu
from jax.experimental.pallas import tpu_sc as plsc

_V = 100000
_E = 64
_B = 1024
_L = 16
_NC, _NS = 2, 16
_NW = _NC * _NS
_BPW = _B // _NW
_NRING = 8


def _rsqrt(x):
    i = lax.bitcast_convert_type(x, jnp.int32)
    y = lax.bitcast_convert_type(jnp.int32(0x5F3759DF) - (i >> 1), jnp.float32)
    for _ in range(3):
        y = y * (1.5 - 0.5 * x * y * y)
    return y


def _body(x_hbm, w_hbm, b_hbm, out_hbm, x_v, blk_v, b_v, out_v, *sems):
    wid = lax.axis_index("s") * _NC + lax.axis_index("c")
    base = wid * _BPW

    pltpu.sync_copy(x_hbm.at[pl.ds(base, _BPW)], x_v)
    pltpu.sync_copy(b_hbm, b_v)

    xv0 = x_v[pl.ds(0, _L)]
    xv1 = x_v[pl.ds(_L, _L)]
    xs = [xv0[i] for i in range(_L)] + [xv1[i] for i in range(_L)]
    bv = [b_v[pl.ds(k * _L, _L)] for k in range(_E // _L)]
    iota = lax.iota(jnp.int32, _L)

    def fire(i):
        ci = pl.multiple_of((xs[i] >> 7) << 7, 128)
        return pltpu.async_copy(
            w_hbm.at[:, pl.ds(ci, 128)], blk_v.at[i % _NRING], sems[i % _NRING]
        )

    copies = {}
    for i in range(_NRING):
        copies[i] = fire(i)

    for i in range(_BPW):
        copies[i].wait()
        s = i % _NRING
        li = jnp.full((_L,), xs[i] & 127, jnp.int32)
        si = jnp.full((_L,), s, jnp.int32)
        ck = [
            plsc.load_gather(blk_v, [si, k * _L + iota, li]) + bv[k]
            for k in range(_E // _L)
        ]
        if i + _NRING < _BPW:
            copies[i + _NRING] = fire(i + _NRING)
        sq = (ck[0] * ck[0] + ck[1] * ck[1]) + (ck[2] * ck[2] + ck[3] * ck[3])
        ss = lax.reduce_sum_p.bind(sq, axes=(0,))
        r = _rsqrt(jnp.maximum(ss, 1e-24))
        ii = jnp.full((_L,), i, jnp.int32)
        for k in range(_E // _L):
            plsc.store_scatter(out_v, [ii, k * _L + iota], ck[k] * r)

    pltpu.sync_copy(out_v, out_hbm.at[pl.ds(base, _BPW)])


@jax.jit
def _road2vec_sc(x, w, b):
    mesh = plsc.VectorSubcoreMesh(core_axis_name="c", subcore_axis_name="s")
    return pl.kernel(
        _body,
        mesh=mesh,
        compiler_params=pltpu.CompilerParams(needs_layout_passes=False),
        out_type=jax.ShapeDtypeStruct((_B, _E), jnp.float32),
        scratch_types=[
            pltpu.VMEM((_BPW,), jnp.int32),
            pltpu.VMEM((_NRING, _E, 128), jnp.float32),
            pltpu.VMEM((_E,), jnp.float32),
            pltpu.VMEM((_BPW, _E), jnp.float32),
        ]
        + [pltpu.SemaphoreType.DMA] * _NRING,
    )(x, w, b)


def kernel(x, W, b):
    return _road2vec_sc(x.astype(jnp.int32), W, b)
